# Initial kernel scaffold; baseline (speedup 1.0000x reference)
import jax
import jax.numpy as jnp
from jax.experimental import pallas as pl


def kernel(embeddings, row_distances, col_distances):
    raise NotImplementedError("write your pallas kernel here")


# speedup vs baseline: 1.8583x; 1.8583x over previous
import functools

import jax
import jax.numpy as jnp
from jax import lax
from jax.experimental import pallas as pl
from jax.experimental.pallas import tpu as pltpu
from jax.experimental.pallas import tpu_sc as plsc

HEAD_DIM = 64
TABLE_ROWS = 15 * 15
BATCH = 64 * 64

_info = plsc.get_sparse_core_info()
_NC, _NS, _L = _info.num_cores, _info.num_subcores, _info.num_lanes
_NW = _NC * _NS
_B_PER_W = BATCH // _NW


def _sc_gather(table, rows, cols):
    mesh = plsc.VectorSubcoreMesh(core_axis_name="c", subcore_axis_name="s")

    @functools.partial(
        pl.kernel,
        mesh=mesh,
        out_type=jax.ShapeDtypeStruct((BATCH, HEAD_DIM), jnp.float32),
        compiler_params=pltpu.CompilerParams(use_tc_tiling_on_sc=False),
        scratch_types=[
            pltpu.VMEM((_B_PER_W,), jnp.int32),
            pltpu.VMEM((_B_PER_W,), jnp.int32),
            pltpu.VMEM((_B_PER_W,), jnp.int32),
            pltpu.VMEM((_B_PER_W, HEAD_DIM), jnp.float32),
            pltpu.SemaphoreType.DMA,
        ],
    )
    def k(table_hbm, row_hbm, col_hbm, out_hbm, row_v, col_v, idx_v, rows_v, sem):
        wid = lax.axis_index("s") * _NC + lax.axis_index("c")
        base = wid * _B_PER_W
        pltpu.sync_copy(row_hbm.at[pl.ds(base, _B_PER_W)], row_v)
        pltpu.sync_copy(col_hbm.at[pl.ds(base, _B_PER_W)], col_v)
        for i in range(_B_PER_W // _L):
            sl = pl.ds(i * _L, _L)
            idx_v[sl] = row_v[sl] * 15 + col_v[sl]
        pltpu.async_copy(table_hbm.at[idx_v], rows_v, sem).wait()
        pltpu.sync_copy(rows_v, out_hbm.at[pl.ds(base, _B_PER_W)])

    return k(table, rows, cols)


def kernel(embeddings, row_distances, col_distances):
    table = embeddings.reshape(TABLE_ROWS, HEAD_DIM)
    rows = row_distances.reshape(BATCH).astype(jnp.int32)
    cols = col_distances.reshape(BATCH).astype(jnp.int32)
    out = _sc_gather(table, rows, cols)
    return out.reshape(64, 64, HEAD_DIM)
